# Initial kernel scaffold; baseline (speedup 1.0000x reference)
import jax
import jax.numpy as jnp
from jax.experimental import pallas as pl


def kernel(x, index):
    raise NotImplementedError("write your pallas kernel here")


# speedup vs baseline: 13.3118x; 13.3118x over previous
import functools

import jax
import jax.numpy as jnp
from jax import lax
from jax.experimental import pallas as pl
from jax.experimental.pallas import tpu as pltpu
from jax.experimental.pallas import tpu_sc as plsc

V = 100000
D = 128
B = 4096 * 50
NW = 32
BPW = B // NW
CLEN = 128
NCHUNK = BPW // CLEN


def _make_gather():
    mesh = plsc.VectorSubcoreMesh(core_axis_name="c", subcore_axis_name="s")

    @functools.partial(
        pl.kernel,
        mesh=mesh,
        out_type=jax.ShapeDtypeStruct((B, D), jnp.float32),
        scratch_types=[
            pltpu.VMEM((NCHUNK, CLEN), jnp.int32),
            pltpu.VMEM((CLEN, D), jnp.float32),
            pltpu.SemaphoreType.DMA,
        ],
    )
    def gather_kernel(x_hbm, idx_hbm, out_hbm, idx_v, buf, gsem):
        wid = lax.axis_index("s") * 2 + lax.axis_index("c")
        base = wid * BPW
        pltpu.sync_copy(idx_hbm.at[wid], idx_v)

        def step(c, carry):
            pltpu.async_copy(x_hbm.at[idx_v.at[c]], buf, gsem).wait()
            pltpu.sync_copy(buf, out_hbm.at[pl.ds(base + c * CLEN, CLEN)])
            return carry

        lax.fori_loop(0, NCHUNK, step, 0)

    return gather_kernel


_gather = _make_gather()


@jax.jit
def kernel(x, index):
    xf = x.reshape(V, D)
    idx = index.astype(jnp.int32).reshape(NW, NCHUNK, CLEN)
    out = _gather(xf, idx)
    return out.reshape(index.shape[0], index.shape[1], 16, 8)


# speedup vs baseline: 13.8342x; 1.0392x over previous
import functools

import jax
import jax.numpy as jnp
from jax import lax
from jax.experimental import pallas as pl
from jax.experimental.pallas import tpu as pltpu
from jax.experimental.pallas import tpu_sc as plsc

V = 100000
D = 128
B = 4096 * 50
NW = 32
BPW = B // NW
CLEN = 128
NCHUNK = BPW // CLEN


NBUF = 5
NGRP = NCHUNK // NBUF


def _make_gather():
    mesh = plsc.VectorSubcoreMesh(core_axis_name="c", subcore_axis_name="s")

    @functools.partial(
        pl.kernel,
        mesh=mesh,
        out_type=jax.ShapeDtypeStruct((B, D), jnp.float32),
        scratch_types=[
            pltpu.VMEM((NCHUNK, CLEN), jnp.int32),
        ]
        + [pltpu.VMEM((CLEN, D), jnp.float32)] * NBUF
        + [pltpu.SemaphoreType.DMA] * (2 * NBUF),
    )
    def gather_kernel(x_hbm, idx_hbm, out_hbm, idx_v, *scr):
        bufs = scr[:NBUF]
        gsem = scr[NBUF:2 * NBUF]
        wsem = scr[2 * NBUF:]
        wid = lax.axis_index("s") * 2 + lax.axis_index("c")
        base = wid * BPW
        pltpu.sync_copy(idx_hbm.at[wid], idx_v)

        for b in range(NBUF):
            pltpu.async_copy(x_hbm.at[idx_v.at[b]], bufs[b], gsem[b])

        def round_(p, carry):
            c0 = p * NBUF
            for b in range(NBUF):
                pltpu.make_async_copy(x_hbm.at[idx_v.at[0]], bufs[b],
                                      gsem[b]).wait()
                pltpu.async_copy(
                    bufs[b], out_hbm.at[pl.ds(base + (c0 + b) * CLEN, CLEN)],
                    wsem[b])
            for b in range(NBUF):
                pltpu.make_async_copy(
                    bufs[b], out_hbm.at[pl.ds(base, CLEN)], wsem[b]).wait()
                pltpu.async_copy(x_hbm.at[idx_v.at[c0 + NBUF + b]], bufs[b],
                                 gsem[b])
            return carry

        lax.fori_loop(0, NGRP - 1, round_, 0)

        c0 = (NGRP - 1) * NBUF
        for b in range(NBUF):
            pltpu.make_async_copy(x_hbm.at[idx_v.at[0]], bufs[b],
                                  gsem[b]).wait()
            pltpu.async_copy(
                bufs[b], out_hbm.at[pl.ds(base + (c0 + b) * CLEN, CLEN)],
                wsem[b])
        for b in range(NBUF):
            pltpu.make_async_copy(
                bufs[b], out_hbm.at[pl.ds(base, CLEN)], wsem[b]).wait()

    return gather_kernel


_gather = _make_gather()


@jax.jit
def kernel(x, index):
    xf = x.reshape(V, D)
    idx = index.astype(jnp.int32).reshape(NW, NCHUNK, CLEN)
    out = _gather(xf, idx)
    return out.reshape(index.shape[0], index.shape[1], 16, 8)


# speedup vs baseline: 41.0561x; 2.9677x over previous
import functools

import jax
import jax.numpy as jnp
from jax import lax
from jax.experimental import pallas as pl
from jax.experimental.pallas import tpu as pltpu
from jax.experimental.pallas import tpu_sc as plsc

V = 100000
P = 128
NB = 4096
S = 50
NW = 32
PPT = P // NW
L = 16
UNROLL = 8
NV = NB // (L * UNROLL)


def _make_gather():
    mesh = plsc.VectorSubcoreMesh(core_axis_name="c", subcore_axis_name="s")

    @functools.partial(
        pl.kernel,
        mesh=mesh,
        compiler_params=pltpu.CompilerParams(needs_layout_passes=False),
        out_type=jax.ShapeDtypeStruct((S * P, NB), jnp.float32),
        scratch_types=[
            pltpu.VMEM((V,), jnp.float32),
            pltpu.VMEM((NB,), jnp.int32),
            pltpu.VMEM((NB,), jnp.int32),
            pltpu.VMEM((NB,), jnp.float32),
            pltpu.VMEM((NB,), jnp.float32),
            pltpu.SemaphoreType.DMA,
            pltpu.SemaphoreType.DMA,
            pltpu.SemaphoreType.DMA,
            pltpu.SemaphoreType.DMA,
        ],
    )
    def gather_kernel(xT, idxT, outT, plane, ia, ib, oa, ob,
                      sia, sib, soa, sob):
        wid = lax.axis_index("s") * 2 + lax.axis_index("c")

        def gather_row(idxb, outb):
            def body(v, carry):
                for u in range(UNROLL):
                    off = (v * UNROLL + u) * L
                    ids = idxb[pl.ds(off, L)]
                    outb[pl.ds(off, L)] = plsc.load_gather(plane, [ids])
                return carry
            lax.fori_loop(0, NV, body, 0)

        def wait_idx(sem):
            pltpu.make_async_copy(idxT.at[0], ia, sem).wait()

        def wait_out(sem):
            pltpu.make_async_copy(oa, outT.at[0], sem).wait()

        for pi in range(PPT):
            p = wid * PPT + pi
            pltpu.sync_copy(xT.at[p], plane)
            pltpu.async_copy(idxT.at[0], ia, sia)
            pltpu.async_copy(idxT.at[1], ib, sib)

            wait_idx(sia)
            gather_row(ia, oa)
            pltpu.async_copy(oa, outT.at[p], soa)
            pltpu.async_copy(idxT.at[2], ia, sia)
            wait_idx(sib)
            gather_row(ib, ob)
            pltpu.async_copy(ob, outT.at[P + p], sob)
            pltpu.async_copy(idxT.at[3], ib, sib)

            def pair(g, carry):
                s0 = 2 * g
                wait_idx(sia)
                wait_out(soa)
                gather_row(ia, oa)
                pltpu.async_copy(oa, outT.at[s0 * P + p], soa)
                pltpu.async_copy(idxT.at[s0 + 2], ia, sia)
                wait_idx(sib)
                wait_out(sob)
                gather_row(ib, ob)
                pltpu.async_copy(ob, outT.at[(s0 + 1) * P + p], sob)
                pltpu.async_copy(idxT.at[s0 + 3], ib, sib)
                return carry

            lax.fori_loop(1, S // 2 - 1, pair, 0)

            wait_idx(sia)
            wait_out(soa)
            gather_row(ia, oa)
            pltpu.async_copy(oa, outT.at[(S - 2) * P + p], soa)
            wait_idx(sib)
            wait_out(sob)
            gather_row(ib, ob)
            pltpu.async_copy(ob, outT.at[(S - 1) * P + p], sob)
            wait_out(soa)
            wait_out(sob)

    return gather_kernel


_gather = _make_gather()


@jax.jit
def kernel(x, index):
    b, s = index.shape
    xT = x.transpose(1, 2, 0).reshape(P, V)
    idxT = index.astype(jnp.int32).T
    outT = _gather(xT, idxT)
    return outT.reshape(s, 16, 8, b).transpose(3, 0, 1, 2)


# speedup vs baseline: 57.3393x; 1.3966x over previous
import functools

import jax
import jax.numpy as jnp
from jax import lax
from jax.experimental import pallas as pl
from jax.experimental.pallas import tpu as pltpu
from jax.experimental.pallas import tpu_sc as plsc

V = 100000
P = 128
NB = 4096
S = 50
NW = 32
PPT = P // NW
L = 16
UNROLL = 8
NV = NB // (L * UNROLL)


def _make_gather():
    mesh = plsc.VectorSubcoreMesh(core_axis_name="c", subcore_axis_name="s")

    @functools.partial(
        pl.kernel,
        mesh=mesh,
        compiler_params=pltpu.CompilerParams(needs_layout_passes=False),
        out_type=jax.ShapeDtypeStruct((S * P, NB), jnp.float32),
        scratch_types=[
            pltpu.VMEM((V,), jnp.float32),
            pltpu.VMEM((NB,), jnp.int32),
            pltpu.VMEM((NB,), jnp.int32),
            pltpu.VMEM((NB,), jnp.float32),
            pltpu.VMEM((NB,), jnp.float32),
            pltpu.SemaphoreType.DMA,
            pltpu.SemaphoreType.DMA,
            pltpu.SemaphoreType.DMA,
            pltpu.SemaphoreType.DMA,
        ],
    )
    def gather_kernel(xT, idxT, outT, plane, ia, ib, oa, ob,
                      sia, sib, soa, sob):
        wid = lax.axis_index("s") * 2 + lax.axis_index("c")

        def gather_row(idxb, outb):
            @plsc.parallel_loop(0, NB, L, unroll=UNROLL)
            def body(i):
                ids = idxb[pl.ds(i, L)]
                outb[pl.ds(i, L)] = plsc.load_gather(plane, [ids])

        def wait_idx(sem):
            pltpu.make_async_copy(idxT.at[0], ia, sem).wait()

        def wait_out(sem):
            pltpu.make_async_copy(oa, outT.at[0], sem).wait()

        for pi in range(PPT):
            p = wid * PPT + pi
            pltpu.sync_copy(xT.at[p], plane)
            pltpu.async_copy(idxT.at[0], ia, sia)
            pltpu.async_copy(idxT.at[1], ib, sib)

            wait_idx(sia)
            gather_row(ia, oa)
            pltpu.async_copy(oa, outT.at[p], soa)
            pltpu.async_copy(idxT.at[2], ia, sia)
            wait_idx(sib)
            gather_row(ib, ob)
            pltpu.async_copy(ob, outT.at[P + p], sob)
            pltpu.async_copy(idxT.at[3], ib, sib)

            def pair(g, carry):
                s0 = 2 * g
                wait_idx(sia)
                wait_out(soa)
                gather_row(ia, oa)
                pltpu.async_copy(oa, outT.at[s0 * P + p], soa)
                pltpu.async_copy(idxT.at[s0 + 2], ia, sia)
                wait_idx(sib)
                wait_out(sob)
                gather_row(ib, ob)
                pltpu.async_copy(ob, outT.at[(s0 + 1) * P + p], sob)
                pltpu.async_copy(idxT.at[s0 + 3], ib, sib)
                return carry

            lax.fori_loop(1, S // 2 - 1, pair, 0)

            wait_idx(sia)
            wait_out(soa)
            gather_row(ia, oa)
            pltpu.async_copy(oa, outT.at[(S - 2) * P + p], soa)
            wait_idx(sib)
            wait_out(sob)
            gather_row(ib, ob)
            pltpu.async_copy(ob, outT.at[(S - 1) * P + p], sob)
            wait_out(soa)
            wait_out(sob)

    return gather_kernel


_gather = _make_gather()


@jax.jit
def kernel(x, index):
    b, s = index.shape
    xT = x.transpose(1, 2, 0).reshape(P, V)
    idxT = index.astype(jnp.int32).T
    outT = _gather(xT, idxT)
    return outT.reshape(s, 16, 8, b).transpose(3, 0, 1, 2)
